# initial kernel scaffold (unmeasured)
import jax
import jax.numpy as jnp
from jax import lax
from jax.experimental import pallas as pl
from jax.experimental.pallas import tpu as pltpu


def kernel(
    x,
):
    def body(*refs):
        pass

    out_shape = jax.ShapeDtypeStruct(..., jnp.float32)
    return pl.pallas_call(body, out_shape=out_shape)(...)


# baseline (device time: 15445 ns/iter reference)
import jax
import jax.numpy as jnp
from jax import lax
from jax.experimental import pallas as pl
from jax.experimental.pallas import tpu as pltpu

N_DEV = 4


def _bitonic_sort_rows(v, m_tot, n):
    idx = lax.broadcasted_iota(jnp.int32, (m_tot, n), 0)
    k = 2
    while k <= m_tot:
        j = k // 2
        while j >= 1:
            bitj = (idx & j) != 0
            partner = jnp.where(
                bitj, pltpu.roll(v, j, 0), pltpu.roll(v, m_tot - j, 0)
            )
            asc = (idx & k) == 0
            take_min = jnp.logical_xor(asc, bitj)
            v = jnp.where(
                take_min, jnp.minimum(v, partner), jnp.maximum(v, partner)
            )
            j //= 2
        k *= 2
    return v


def kernel(x):
    m_per, n = x.shape
    m_tot = N_DEV * m_per

    def body(x_ref, out_ref, gather_ref, send_sems, recv_sems):
        my = lax.axis_index("i")
        left = (my - 1) % N_DEV
        right = (my + 1) % N_DEV

        barrier_sem = pltpu.get_barrier_semaphore()
        for nbr in [left, right]:
            pl.semaphore_signal(
                barrier_sem, inc=1,
                device_id=(nbr,), device_id_type=pl.DeviceIdType.MESH,
            )
        pl.semaphore_wait(barrier_sem, 2)

        gather_ref[pl.ds(my * m_per, m_per), :] = x_ref[:, :]

        for h in range(N_DEV - 1):
            slot = ((my - h) % N_DEV) * m_per
            rdma = pltpu.make_async_remote_copy(
                src_ref=gather_ref.at[pl.ds(slot, m_per)],
                dst_ref=gather_ref.at[pl.ds(slot, m_per)],
                send_sem=send_sems.at[h],
                recv_sem=recv_sems.at[h],
                device_id=(right,),
                device_id_type=pl.DeviceIdType.MESH,
            )
            rdma.start()
            rdma.wait()

        v = _bitonic_sort_rows(gather_ref[:, :], m_tot, n)
        gather_ref[:, :] = v
        out_ref[:, :] = gather_ref[pl.ds(my * m_per, m_per), :]

    return pl.pallas_call(
        body,
        out_shape=jax.ShapeDtypeStruct((m_per, n), x.dtype),
        in_specs=[pl.BlockSpec(memory_space=pltpu.VMEM)],
        out_specs=pl.BlockSpec(memory_space=pltpu.VMEM),
        scratch_shapes=[
            pltpu.VMEM((m_tot, n), x.dtype),
            pltpu.SemaphoreType.DMA((N_DEV - 1,)),
            pltpu.SemaphoreType.DMA((N_DEV - 1,)),
        ],
        compiler_params=pltpu.CompilerParams(collective_id=0),
    )(x)


# device time: 9354 ns/iter; 1.6512x vs baseline; 1.6512x over previous
import jax
import jax.numpy as jnp
from jax import lax
from jax.experimental import pallas as pl
from jax.experimental.pallas import tpu as pltpu

N_DEV = 4


def _compare_exchange(v, idx, j, k, m, flip):
    bitj = (idx & j) != 0
    partner = jnp.where(bitj, pltpu.roll(v, j, 0), pltpu.roll(v, m - j, 0))
    asc = (idx & k) == 0
    take_min = jnp.logical_xor(jnp.logical_xor(asc, bitj), flip)
    return jnp.where(take_min, jnp.minimum(v, partner), jnp.maximum(v, partner))


def kernel(x):
    m_per, n = x.shape
    m_tot = N_DEV * m_per

    def body(x_ref, out_ref, gather_ref, send_sems, recv_sems):
        my = lax.axis_index("i")

        barrier_sem = pltpu.get_barrier_semaphore()
        for d in range(1, N_DEV):
            pl.semaphore_signal(
                barrier_sem, inc=1,
                device_id=((my + d) % N_DEV,),
                device_id_type=pl.DeviceIdType.MESH,
            )
        pl.semaphore_wait(barrier_sem, N_DEV - 1)

        v = x_ref[:, :]
        lidx = lax.broadcasted_iota(jnp.int32, (m_per, n), 0)
        desc = (my % 2) == 1
        k = 2
        while k <= m_per:
            j = k // 2
            while j >= 1:
                v = _compare_exchange(v, lidx, j, k, m_per, desc)
                j //= 2
            k *= 2
        gather_ref[pl.ds(my * m_per, m_per), :] = v

        rdmas = []
        for d in range(1, N_DEV):
            rdma = pltpu.make_async_remote_copy(
                src_ref=gather_ref.at[pl.ds(my * m_per, m_per)],
                dst_ref=gather_ref.at[pl.ds(my * m_per, m_per)],
                send_sem=send_sems.at[d - 1],
                recv_sem=recv_sems.at[d - 1],
                device_id=((my + d) % N_DEV,),
                device_id_type=pl.DeviceIdType.MESH,
            )
            rdma.start()
            rdmas.append(rdma)
        for rdma in rdmas:
            rdma.wait()

        v = gather_ref[:, :]
        idx = lax.broadcasted_iota(jnp.int32, (m_tot, n), 0)
        k = 2 * m_per
        while k <= m_tot:
            j = k // 2
            while j >= 1:
                v = _compare_exchange(v, idx, j, k, m_tot, False)
                j //= 2
            k *= 2
        gather_ref[:, :] = v
        out_ref[:, :] = gather_ref[pl.ds(my * m_per, m_per), :]

    return pl.pallas_call(
        body,
        out_shape=jax.ShapeDtypeStruct((m_per, n), x.dtype),
        in_specs=[pl.BlockSpec(memory_space=pltpu.VMEM)],
        out_specs=pl.BlockSpec(memory_space=pltpu.VMEM),
        scratch_shapes=[
            pltpu.VMEM((m_tot, n), x.dtype),
            pltpu.SemaphoreType.DMA((N_DEV - 1,)),
            pltpu.SemaphoreType.DMA((N_DEV - 1,)),
        ],
        compiler_params=pltpu.CompilerParams(collective_id=0),
    )(x)


# device time: 8155 ns/iter; 1.8939x vs baseline; 1.1470x over previous
import jax
import jax.numpy as jnp
from jax import lax
from jax.experimental import pallas as pl
from jax.experimental.pallas import tpu as pltpu

N_DEV = 4
HALF = 64


def _ce(v, j, asc, flip):
    m_rows = v.shape[0]
    if j == HALF:
        lidx = lax.broadcasted_iota(jnp.int32, v.shape, 1)
        bitj = (lidx & HALF) != 0
        partner = pltpu.roll(v, HALF, 1)
    else:
        jr = j if j < HALF else j // 2
        ridx = lax.broadcasted_iota(jnp.int32, v.shape, 0)
        bitj = (ridx & jr) != 0
        partner = jnp.where(
            bitj, pltpu.roll(v, jr, 0), pltpu.roll(v, m_rows - jr, 0)
        )
    take_min = jnp.logical_xor(jnp.logical_xor(asc, bitj), flip)
    return jnp.where(take_min, jnp.minimum(v, partner), jnp.maximum(v, partner))


def kernel(x):
    m_per, n = x.shape
    assert m_per == 2 * HALF and n == HALF
    m_rows = N_DEV * HALF

    def body(x_ref, out_ref, gather_ref, send_sems, recv_sems):
        my = lax.axis_index("i")

        barrier_sem = pltpu.get_barrier_semaphore()
        for d in range(1, N_DEV):
            pl.semaphore_signal(
                barrier_sem, inc=1,
                device_id=((my + d) % N_DEV,),
                device_id_type=pl.DeviceIdType.MESH,
            )
        pl.semaphore_wait(barrier_sem, N_DEV - 1)

        xv = x_ref[:, :]
        v = jnp.concatenate([xv[:HALF, :], xv[HALF:, :]], axis=1)

        desc = (my % 2) == 1
        ridx = lax.broadcasted_iota(jnp.int32, v.shape, 0)
        lidx = lax.broadcasted_iota(jnp.int32, v.shape, 1)
        k = 2
        while k <= m_per:
            if k < HALF:
                asc = (ridx & k) == 0
            elif k == HALF:
                asc = (lidx & HALF) == 0
            else:
                asc = True
            j = k // 2
            while j >= 1:
                v = _ce(v, j, asc, desc)
                j //= 2
            k *= 2
        gather_ref[pl.ds(my * HALF, HALF), :] = v

        rdmas = []
        for d in range(1, N_DEV):
            rdma = pltpu.make_async_remote_copy(
                src_ref=gather_ref.at[pl.ds(my * HALF, HALF)],
                dst_ref=gather_ref.at[pl.ds(my * HALF, HALF)],
                send_sem=send_sems.at[d - 1],
                recv_sem=recv_sems.at[d - 1],
                device_id=((my + d) % N_DEV,),
                device_id_type=pl.DeviceIdType.MESH,
            )
            rdma.start()
            rdmas.append(rdma)
        for rdma in rdmas:
            rdma.wait_send()

        def merge_round_256(slab, flip):
            for j in (128, 64, 32, 16, 8, 4, 2, 1):
                slab = _ce(slab, j, True, flip)
            return slab

        B = my // 2
        rdmas[0].wait_recv()
        rdmas[2].wait_recv()
        near = gather_ref[pl.ds(B * 2 * HALF, 2 * HALF), :]
        near = merge_round_256(near, B == 1)
        gather_ref[pl.ds(B * 2 * HALF, 2 * HALF), :] = near

        rdmas[1].wait_recv()
        far = gather_ref[pl.ds((1 - B) * 2 * HALF, 2 * HALF), :]
        far = merge_round_256(far, B == 0)
        gather_ref[pl.ds((1 - B) * 2 * HALF, 2 * HALF), :] = far

        v = gather_ref[:, :]
        v = _ce(v, 256, True, False)
        v = _ce(v, 128, True, False)
        gather_ref[:, :] = v

        w = gather_ref[pl.ds(my * HALF, HALF), :]
        for j in (64, 32, 16, 8, 4, 2, 1):
            w = _ce(w, j, True, False)

        out_ref[0:HALF, :] = w[:, :HALF]
        out_ref[HALF : 2 * HALF, :] = w[:, HALF:]

    return pl.pallas_call(
        body,
        out_shape=jax.ShapeDtypeStruct((m_per, n), x.dtype),
        in_specs=[pl.BlockSpec(memory_space=pltpu.VMEM)],
        out_specs=pl.BlockSpec(memory_space=pltpu.VMEM),
        scratch_shapes=[
            pltpu.VMEM((m_rows, 2 * HALF), x.dtype),
            pltpu.SemaphoreType.DMA((N_DEV - 1,)),
            pltpu.SemaphoreType.DMA((N_DEV - 1,)),
        ],
        compiler_params=pltpu.CompilerParams(collective_id=0),
    )(x)
